# Initial kernel scaffold; baseline (speedup 1.0000x reference)
import jax
import jax.numpy as jnp
from jax.experimental import pallas as pl


def kernel(g_pred_logits, a_pred_logits, a_pred_boxes, assign_src, assign_tgt):
    raise NotImplementedError("write your pallas kernel here")


# speedup vs baseline: 1.0478x; 1.0478x over previous
import functools

import jax
import jax.numpy as jnp
from jax import lax
from jax.experimental import pallas as pl

_C = 1203
_Q = 900
_ALPHA = 0.25


def _body(a_ref, g_ref, src_ref, tgt_ref, out_ref):
    b = pl.program_id(0)
    nb = pl.num_programs(0)

    av = a_ref[0]
    gv = g_ref[0]
    s_row = src_ref[0]
    t_row = tgt_ref[0]

    iota_c = lax.broadcasted_iota(jnp.int32, (_Q, _C), 1)

    m = jnp.max(av, axis=1, keepdims=True)
    lab_col = jnp.min(jnp.where(av == m, iota_c, _C), axis=1, keepdims=True)

    iota_q0 = lax.broadcasted_iota(jnp.int32, (_Q, _Q), 0)
    n_mask = iota_q0 == t_row
    lt_row = jnp.sum(jnp.where(n_mask, lab_col, 0), axis=0, keepdims=True)

    m_mask = iota_q0 == s_row
    k_col = jnp.sum(jnp.where(m_mask, lt_row, 0), axis=1, keepdims=True)

    x = gv
    e = jnp.exp(-jnp.abs(x))
    lg = jnp.log1p(e)
    sp_pos = jnp.maximum(x, 0.0) + lg
    r = 1.0 / (1.0 + e)
    sig = jnp.where(x >= 0, r, 1.0 - r)
    l0 = (1.0 - _ALPHA) * sp_pos * sig * sig

    sel = iota_c == k_col
    xk = jnp.sum(jnp.where(sel, x, 0.0), axis=1, keepdims=True)
    ek = jnp.exp(-jnp.abs(xk))
    sp_neg_k = jnp.maximum(xk, 0.0) + jnp.log1p(ek) - xk
    rk = 1.0 / (1.0 + ek)
    sigk = jnp.where(xk >= 0, rk, 1.0 - rk)
    omk = 1.0 - sigk
    l1k = _ALPHA * sp_neg_k * omk * omk
    l0k = (1.0 - _ALPHA) * (jnp.maximum(xk, 0.0) + jnp.log1p(ek)) * sigk * sigk

    contrib = jnp.sum(l0) + jnp.sum(l1k - l0k)

    @pl.when(b == 0)
    def _init():
        out_ref[...] = jnp.zeros((1, 1), jnp.float32)

    out_ref[...] += jnp.full((1, 1), contrib, jnp.float32)

    @pl.when(b == nb - 1)
    def _finish():
        out_ref[...] = out_ref[...] / (nb * _Q)


@jax.jit
def _run(g_pred_logits, a_pred_logits, assign_src, assign_tgt):
    B, Q, C = g_pred_logits.shape
    src3 = assign_src.reshape(B, 1, Q)
    tgt3 = assign_tgt.reshape(B, 1, Q)
    out = pl.pallas_call(
        _body,
        grid=(B,),
        in_specs=[
            pl.BlockSpec((1, Q, C), lambda b: (b, 0, 0)),
            pl.BlockSpec((1, Q, C), lambda b: (b, 0, 0)),
            pl.BlockSpec((1, 1, Q), lambda b: (b, 0, 0)),
            pl.BlockSpec((1, 1, Q), lambda b: (b, 0, 0)),
        ],
        out_specs=pl.BlockSpec((1, 1), lambda b: (0, 0)),
        out_shape=jax.ShapeDtypeStruct((1, 1), jnp.float32),
    )(a_pred_logits, g_pred_logits, src3, tgt3)
    return out[0, 0]


def kernel(g_pred_logits, a_pred_logits, a_pred_boxes, assign_src, assign_tgt):
    del a_pred_boxes
    return _run(g_pred_logits, a_pred_logits, assign_src, assign_tgt)


# speedup vs baseline: 1.1053x; 1.0549x over previous
import jax
import jax.numpy as jnp
from jax import lax
from jax.experimental import pallas as pl

_C = 1203
_Q = 900
_ALPHA = 0.25
_LOG2E = 1.4426950408889634
_LN2 = 0.6931471805599453


def _body(a_ref, g_ref, src_ref, tgt_ref, out_ref):
    b = pl.program_id(0)
    nb = pl.num_programs(0)

    av = a_ref[0]
    gv = g_ref[0]
    s_row = src_ref[0]
    t_row = tgt_ref[0]

    iota_c = lax.broadcasted_iota(jnp.int32, (_Q, _C), 1)

    m = jnp.max(av, axis=1, keepdims=True)
    lab_col = jnp.min(jnp.where(av == m, iota_c, _C), axis=1, keepdims=True)

    iota_q0 = lax.broadcasted_iota(jnp.int32, (_Q, _Q), 0)
    n_mask = iota_q0 == t_row
    lt_row = jnp.sum(jnp.where(n_mask, lab_col, 0), axis=0, keepdims=True)

    m_mask = iota_q0 == s_row
    k_col = jnp.sum(jnp.where(m_mask, lt_row, 0), axis=1, keepdims=True)

    x = gv
    nax = -jnp.abs(x)
    e = jnp.exp2(nax * _LOG2E)
    u = jnp.log2(1.0 + e)
    sp = jnp.maximum(x, 0.0) + u * _LN2
    s2 = jnp.exp2(jnp.minimum(x, 0.0) * (2.0 * _LOG2E) - 2.0 * u)
    l0s = jnp.sum(sp * s2)

    sel = iota_c == k_col
    xk = jnp.sum(jnp.where(sel, x, 0.0), axis=1, keepdims=True)
    ek = jnp.exp2(-jnp.abs(xk) * _LOG2E)
    uk = jnp.log2(1.0 + ek)
    sp_p = jnp.maximum(xk, 0.0) + uk * _LN2
    sp_n = sp_p - xk
    s2_p = jnp.exp2(jnp.minimum(xk, 0.0) * (2.0 * _LOG2E) - 2.0 * uk)
    s2_n = jnp.exp2(jnp.minimum(-xk, 0.0) * (2.0 * _LOG2E) - 2.0 * uk)
    corr = jnp.sum(_ALPHA * sp_n * s2_n - (1.0 - _ALPHA) * sp_p * s2_p)

    contrib = (1.0 - _ALPHA) * l0s + corr

    @pl.when(b == 0)
    def _init():
        out_ref[...] = jnp.zeros((1, 1), jnp.float32)

    out_ref[...] += jnp.full((1, 1), contrib, jnp.float32)

    @pl.when(b == nb - 1)
    def _finish():
        out_ref[...] = out_ref[...] / (nb * _Q)


@jax.jit
def _run(g_pred_logits, a_pred_logits, assign_src, assign_tgt):
    B, Q, C = g_pred_logits.shape
    src3 = assign_src.reshape(B, 1, Q)
    tgt3 = assign_tgt.reshape(B, 1, Q)
    out = pl.pallas_call(
        _body,
        grid=(B,),
        in_specs=[
            pl.BlockSpec((1, Q, C), lambda b: (b, 0, 0)),
            pl.BlockSpec((1, Q, C), lambda b: (b, 0, 0)),
            pl.BlockSpec((1, 1, Q), lambda b: (b, 0, 0)),
            pl.BlockSpec((1, 1, Q), lambda b: (b, 0, 0)),
        ],
        out_specs=pl.BlockSpec((1, 1), lambda b: (0, 0)),
        out_shape=jax.ShapeDtypeStruct((1, 1), jnp.float32),
    )(a_pred_logits, g_pred_logits, src3, tgt3)
    return out[0, 0]


def kernel(g_pred_logits, a_pred_logits, a_pred_boxes, assign_src, assign_tgt):
    del a_pred_boxes
    return _run(g_pred_logits, a_pred_logits, assign_src, assign_tgt)


# speedup vs baseline: 1.1249x; 1.0177x over previous
import jax
import jax.numpy as jnp
from jax import lax
from jax.experimental import pallas as pl

_C = 1203
_Q = 900
_ALPHA = 0.25
_LOG2E = 1.4426950408889634
_LN2 = 0.6931471805599453


def _body(a_ref, g_ref, src_ref, tgt_ref, out_ref):
    b = pl.program_id(0)
    nb = pl.num_programs(0)

    av = a_ref[0]
    gv = g_ref[0]
    s_row = src_ref[b]
    t_row = tgt_ref[b]

    iota_c = lax.broadcasted_iota(jnp.int32, (_Q, _C), 1)

    m = jnp.max(av, axis=1, keepdims=True)
    lab_col = jnp.min(jnp.where(av == m, iota_c, _C), axis=1, keepdims=True)

    iota_q0 = lax.broadcasted_iota(jnp.int32, (_Q, _Q), 0)
    n_mask = iota_q0 == t_row
    lt_row = jnp.sum(jnp.where(n_mask, lab_col, 0), axis=0, keepdims=True)

    m_mask = iota_q0 == s_row
    k_col = jnp.sum(jnp.where(m_mask, lt_row, 0), axis=1, keepdims=True)

    y = gv * _LOG2E
    e = jnp.exp2(jnp.minimum(y, -y))
    u = jnp.log2(1.0 + e)
    sp = jnp.maximum(y, 0.0) + u
    z = jnp.minimum(y, 0.0) - u
    s2 = jnp.exp2(z + z)
    l0s = jnp.sum(sp * s2)

    sel = iota_c == k_col
    yk = jnp.sum(jnp.where(sel, y, 0.0), axis=1, keepdims=True)
    ek = jnp.exp2(jnp.minimum(yk, -yk))
    uk = jnp.log2(1.0 + ek)
    sp_p = jnp.maximum(yk, 0.0) + uk
    sp_n = sp_p - yk
    zp = jnp.minimum(yk, 0.0) - uk
    zn = jnp.minimum(-yk, 0.0) - uk
    s2_p = jnp.exp2(zp + zp)
    s2_n = jnp.exp2(zn + zn)
    corr = jnp.sum(_ALPHA * sp_n * s2_n - (1.0 - _ALPHA) * sp_p * s2_p)

    contrib = ((1.0 - _ALPHA) * l0s + corr) * _LN2

    @pl.when(b == 0)
    def _init():
        out_ref[...] = jnp.zeros((1, 1), jnp.float32)

    out_ref[...] += jnp.full((1, 1), contrib, jnp.float32)

    @pl.when(b == nb - 1)
    def _finish():
        out_ref[...] = out_ref[...] / (nb * _Q)


@jax.jit
def _run(g_pred_logits, a_pred_logits, assign_src, assign_tgt):
    B, Q, C = g_pred_logits.shape
    src3 = assign_src.reshape(B, 1, Q)
    tgt3 = assign_tgt.reshape(B, 1, Q)
    out = pl.pallas_call(
        _body,
        grid=(B,),
        in_specs=[
            pl.BlockSpec((1, Q, C), lambda b: (b, 0, 0)),
            pl.BlockSpec((1, Q, C), lambda b: (b, 0, 0)),
            pl.BlockSpec((B, 1, Q), lambda b: (0, 0, 0)),
            pl.BlockSpec((B, 1, Q), lambda b: (0, 0, 0)),
        ],
        out_specs=pl.BlockSpec((1, 1), lambda b: (0, 0)),
        out_shape=jax.ShapeDtypeStruct((1, 1), jnp.float32),
    )(a_pred_logits, g_pred_logits, src3, tgt3)
    return out[0, 0]


def kernel(g_pred_logits, a_pred_logits, a_pred_boxes, assign_src, assign_tgt):
    del a_pred_boxes
    return _run(g_pred_logits, a_pred_logits, assign_src, assign_tgt)
